# baseline (speedup 1.0000x reference)
import jax
import jax.numpy as jnp
from jax.experimental import pallas as pl
from jax.experimental.pallas import tpu as pltpu

N, C, H, W = 32, 256, 56, 56
G = 16
EPS = 1e-05
N_ITER = 10
HW = H * W
M2 = N * HW
BMS = 25088
NBS = M2 // BMS
BM = 12544
NBLK = M2 // BM
M_TOT = N * (C // G) * HW


def _stats_kernel(x_ref, q_ref, s_ref):
    i = pl.program_id(0)

    @pl.when(i == 0)
    def _():
        q_ref[...] = jnp.zeros_like(q_ref)
        s_ref[...] = jnp.zeros_like(s_ref)

    a = x_ref[...]
    ab = a.astype(jnp.bfloat16)
    q = jax.lax.dot_general(ab, ab, (((0,), (0,)), ((), ())),
                            preferred_element_type=jnp.float32)
    q_ref[0] += q
    s_ref[0] += jnp.sum(a, axis=0, keepdims=True)


def _finish_body(q_ref, s_ref, w_ref, b_ref, d_ref, o_ref):
    Q = q_ref[0]
    s_row = s_ref[0]

    ri = jax.lax.broadcasted_iota(jnp.int32, (C, C), 0)
    ci = jax.lax.broadcasted_iota(jnp.int32, (C, C), 1)
    bd = ((ri // G) == (ci // G)).astype(jnp.float32)
    gi = jax.lax.broadcasted_iota(jnp.int32, (G, C), 0)
    cg = jax.lax.broadcasted_iota(jnp.int32, (G, C), 1)
    sel = ((cg % G) == gi).astype(jnp.float32)
    r2 = jax.lax.broadcasted_iota(jnp.int32, (C, G), 0)
    g2 = jax.lax.broadcasted_iota(jnp.int32, (C, G), 1)
    sel_t = ((r2 % G) == g2).astype(jnp.float32)
    eye = (jax.lax.broadcasted_iota(jnp.int32, (G, G), 0)
           == jax.lax.broadcasted_iota(jnp.int32, (G, G), 1)
           ).astype(jnp.float32)

    inv_m = jnp.float32(1.0 / M_TOT)
    mean_col = jax.lax.dot_general(sel, s_row, (((1,), (1,)), ((), ())),
                                   preferred_element_type=jnp.float32) * inv_m
    mean_row = jnp.dot(s_row, sel_t,
                       preferred_element_type=jnp.float32) * inv_m
    p16 = jnp.dot(jnp.dot(sel, Q * bd, preferred_element_type=jnp.float32),
                  sel_t, preferred_element_type=jnp.float32)
    cov = p16 * inv_m - mean_col * mean_row + EPS * eye

    norm_a = jnp.sqrt(jnp.sum(cov * cov))
    ymat = cov / norm_a
    zmat = eye
    for _ in range(N_ITER):
        tmat = 0.5 * (3.0 * eye
                      - jnp.dot(zmat, ymat, preferred_element_type=jnp.float32))
        ymat = jnp.dot(ymat, tmat, preferred_element_type=jnp.float32)
        zmat = jnp.dot(tmat, zmat, preferred_element_type=jnp.float32)
    decorr = zmat / jnp.sqrt(norm_a)

    w_row = w_ref[...]
    b_row = b_ref[...]
    dt16 = jax.lax.dot_general(eye, decorr, (((1,), (1,)), ((), ())),
                               preferred_element_type=jnp.float32)
    dt_tile = jnp.dot(jnp.dot(sel_t, dt16, preferred_element_type=jnp.float32),
                      sel, preferred_element_type=jnp.float32)
    d_ref[...] = dt_tile * bd * w_row
    dm = jnp.dot(decorr, mean_col, preferred_element_type=jnp.float32)
    dm_row = jax.lax.dot_general(dm, sel, (((0,), (0,)), ((), ())),
                                 preferred_element_type=jnp.float32)
    o_ref[...] = b_row - w_row * dm_row


def _apply_kernel(x_ref, q_ref, s_ref, w_ref, b_ref, y_ref, d_s, o_s):
    i = pl.program_id(0)

    @pl.when(i == 0)
    def _():
        _finish_body(q_ref, s_ref, w_ref, b_ref, d_s, o_s)

    y_ref[...] = (jnp.dot(x_ref[...], d_s[...],
                          preferred_element_type=jnp.float32)
                  + o_s[...])


def kernel(x, weight, bias):
    xp = x.transpose(0, 2, 3, 1).reshape(M2, C)
    w = weight.reshape(1, C)
    b = bias.reshape(1, C)

    qp, sp = pl.pallas_call(
        _stats_kernel,
        grid=(NBS,),
        in_specs=[pl.BlockSpec((BMS, C), lambda i: (i, 0))],
        out_specs=[
            pl.BlockSpec((1, C, C), lambda i: (0, 0, 0)),
            pl.BlockSpec((1, 1, C), lambda i: (0, 0, 0)),
        ],
        out_shape=[
            jax.ShapeDtypeStruct((1, C, C), jnp.float32),
            jax.ShapeDtypeStruct((1, 1, C), jnp.float32),
        ],
        compiler_params=pltpu.CompilerParams(
            dimension_semantics=("arbitrary",),
        ),
        name="decorr_stats",
    )(xp)

    y2d = pl.pallas_call(
        _apply_kernel,
        grid=(NBLK,),
        in_specs=[
            pl.BlockSpec((BM, C), lambda i: (i, 0)),
            pl.BlockSpec((1, C, C), lambda i: (0, 0, 0)),
            pl.BlockSpec((1, 1, C), lambda i: (0, 0, 0)),
            pl.BlockSpec((1, C), lambda i: (0, 0)),
            pl.BlockSpec((1, C), lambda i: (0, 0)),
        ],
        out_specs=pl.BlockSpec((BM, C), lambda i: (i, 0)),
        out_shape=jax.ShapeDtypeStruct((M2, C), jnp.float32),
        scratch_shapes=[
            pltpu.VMEM((C, C), jnp.float32),
            pltpu.VMEM((1, C), jnp.float32),
        ],
        compiler_params=pltpu.CompilerParams(
            dimension_semantics=("arbitrary",),
        ),
        name="decorr_apply",
    )(xp, qp, sp, w, b)

    return y2d.reshape(N, H, W, C).transpose(0, 3, 1, 2)


# speedup vs baseline: 1.0249x; 1.0249x over previous
import jax
import jax.numpy as jnp
from jax.experimental import pallas as pl
from jax.experimental.pallas import tpu as pltpu

N, C, H, W = 32, 256, 56, 56
G = 16
EPS = 1e-05
N_ITER = 10
HW = H * W
M2 = N * HW
BMS = 12544
NBS = M2 // BMS
BM = 12544
NBLK = M2 // BM
M_TOT = N * (C // G) * HW


def _stats_kernel(x_ref, q_ref, s_ref):
    i = pl.program_id(0)

    @pl.when(i == 0)
    def _():
        q_ref[...] = jnp.zeros_like(q_ref)
        s_ref[...] = jnp.zeros_like(s_ref)

    a = x_ref[...]
    ab = a.astype(jnp.bfloat16)
    q = jax.lax.dot_general(ab, ab, (((0,), (0,)), ((), ())),
                            preferred_element_type=jnp.float32)
    q_ref[0] += q
    s_ref[0] += jnp.sum(a, axis=0, keepdims=True)


def _finish_body(q_ref, s_ref, w_ref, b_ref, d_ref, o_ref):
    Q = q_ref[0]
    s_row = s_ref[0]

    ri = jax.lax.broadcasted_iota(jnp.int32, (C, C), 0)
    ci = jax.lax.broadcasted_iota(jnp.int32, (C, C), 1)
    bd = ((ri // G) == (ci // G)).astype(jnp.float32)
    gi = jax.lax.broadcasted_iota(jnp.int32, (G, C), 0)
    cg = jax.lax.broadcasted_iota(jnp.int32, (G, C), 1)
    sel = ((cg % G) == gi).astype(jnp.float32)
    r2 = jax.lax.broadcasted_iota(jnp.int32, (C, G), 0)
    g2 = jax.lax.broadcasted_iota(jnp.int32, (C, G), 1)
    sel_t = ((r2 % G) == g2).astype(jnp.float32)
    eye = (jax.lax.broadcasted_iota(jnp.int32, (G, G), 0)
           == jax.lax.broadcasted_iota(jnp.int32, (G, G), 1)
           ).astype(jnp.float32)

    inv_m = jnp.float32(1.0 / M_TOT)
    mean_col = jax.lax.dot_general(sel, s_row, (((1,), (1,)), ((), ())),
                                   preferred_element_type=jnp.float32) * inv_m
    mean_row = jnp.dot(s_row, sel_t,
                       preferred_element_type=jnp.float32) * inv_m
    p16 = jnp.dot(jnp.dot(sel, Q * bd, preferred_element_type=jnp.float32),
                  sel_t, preferred_element_type=jnp.float32)
    cov = p16 * inv_m - mean_col * mean_row + EPS * eye

    norm_a = jnp.sqrt(jnp.sum(cov * cov))
    ymat = cov / norm_a
    zmat = eye
    for _ in range(N_ITER):
        tmat = 0.5 * (3.0 * eye
                      - jnp.dot(zmat, ymat, preferred_element_type=jnp.float32))
        ymat = jnp.dot(ymat, tmat, preferred_element_type=jnp.float32)
        zmat = jnp.dot(tmat, zmat, preferred_element_type=jnp.float32)
    decorr = zmat / jnp.sqrt(norm_a)

    w_row = w_ref[...]
    b_row = b_ref[...]
    dt16 = jax.lax.dot_general(eye, decorr, (((1,), (1,)), ((), ())),
                               preferred_element_type=jnp.float32)
    dt_tile = jnp.dot(jnp.dot(sel_t, dt16, preferred_element_type=jnp.float32),
                      sel, preferred_element_type=jnp.float32)
    d_ref[...] = dt_tile * bd * w_row
    dm = jnp.dot(decorr, mean_col, preferred_element_type=jnp.float32)
    dm_row = jax.lax.dot_general(dm, sel, (((0,), (0,)), ((), ())),
                                 preferred_element_type=jnp.float32)
    o_ref[...] = b_row - w_row * dm_row


def _apply_kernel(x_ref, q_ref, s_ref, w_ref, b_ref, y_ref, d_s, o_s):
    i = pl.program_id(0)

    @pl.when(i == 0)
    def _():
        _finish_body(q_ref, s_ref, w_ref, b_ref, d_s, o_s)

    y_ref[...] = (jnp.dot(x_ref[...], d_s[...],
                          preferred_element_type=jnp.float32)
                  + o_s[...])


def kernel(x, weight, bias):
    xp = x.transpose(0, 2, 3, 1).reshape(M2, C)
    w = weight.reshape(1, C)
    b = bias.reshape(1, C)

    qp, sp = pl.pallas_call(
        _stats_kernel,
        grid=(NBS,),
        in_specs=[pl.BlockSpec((BMS, C), lambda i: (i, 0))],
        out_specs=[
            pl.BlockSpec((1, C, C), lambda i: (0, 0, 0)),
            pl.BlockSpec((1, 1, C), lambda i: (0, 0, 0)),
        ],
        out_shape=[
            jax.ShapeDtypeStruct((1, C, C), jnp.float32),
            jax.ShapeDtypeStruct((1, 1, C), jnp.float32),
        ],
        compiler_params=pltpu.CompilerParams(
            dimension_semantics=("arbitrary",),
        ),
        name="decorr_stats",
    )(xp)

    y2d = pl.pallas_call(
        _apply_kernel,
        grid=(NBLK,),
        in_specs=[
            pl.BlockSpec((BM, C), lambda i: (i, 0)),
            pl.BlockSpec((1, C, C), lambda i: (0, 0, 0)),
            pl.BlockSpec((1, 1, C), lambda i: (0, 0, 0)),
            pl.BlockSpec((1, C), lambda i: (0, 0)),
            pl.BlockSpec((1, C), lambda i: (0, 0)),
        ],
        out_specs=pl.BlockSpec((BM, C), lambda i: (i, 0)),
        out_shape=jax.ShapeDtypeStruct((M2, C), jnp.float32),
        scratch_shapes=[
            pltpu.VMEM((C, C), jnp.float32),
            pltpu.VMEM((1, C), jnp.float32),
        ],
        compiler_params=pltpu.CompilerParams(
            dimension_semantics=("arbitrary",),
        ),
        name="decorr_apply",
    )(xp, qp, sp, w, b)

    return y2d.reshape(N, H, W, C).transpose(0, 3, 1, 2)
